# Initial kernel scaffold; baseline (speedup 1.0000x reference)
import jax
import jax.numpy as jnp
from jax.experimental import pallas as pl


def kernel(voxel_features, coords):
    raise NotImplementedError("write your pallas kernel here")


# speedup vs baseline: 1.1133x; 1.1133x over previous
import functools

import jax
import jax.numpy as jnp
from jax import lax
from jax.experimental import pallas as pl
from jax.experimental.pallas import tpu as pltpu
from jax.experimental.pallas import tpu_sc as plsc

NXK = 512
NYK = 512
NCH = 64
NVOX = 40000
NPAD = 40960
PLANE = NXK * NYK
QUARTER = PLANE // 4
VAL_CHUNK = 20480


def _tc_prep_body(vf_ref, ct_ref, vft_ref, idx_ref):
    vft_ref[...] = vf_ref[...].T
    i = pl.program_id(0)
    ct = ct_ref[:, pl.ds(i * 2048, 2048)]
    idx_ref[...] = ct[0, :] * NYK + ct[1, :]


def _tc_prep(vf, coords_t):
    blk = 2048
    grid = NPAD // blk
    return pl.pallas_call(
        _tc_prep_body,
        grid=(grid,),
        in_specs=[
            pl.BlockSpec((blk, NCH), lambda i: (i, 0)),
            pl.BlockSpec((2, NPAD), lambda i: (0, 0)),
        ],
        out_specs=[
            pl.BlockSpec((NCH, blk), lambda i: (0, i)),
            pl.BlockSpec((blk,), lambda i: (i,)),
        ],
        out_shape=[
            jax.ShapeDtypeStruct((NCH, NPAD), jnp.float32),
            jax.ShapeDtypeStruct((NPAD,), jnp.int32),
        ],
    )(vf, coords_t)


def _sc_scatter_body(vft_hbm, idx_hbm, out_hbm, idx_v, val_v, qbuf):
    wid = lax.axis_index("s") * 2 + lax.axis_index("c")

    pltpu.sync_copy(idx_hbm, idx_v)

    def zero_body(i, _):
        qbuf[pl.ds(i * 16, 16)] = jnp.zeros((16,), jnp.float32)
        return _

    for q in range(4):
        for rc in range(2):
            ch = wid * 2 + rc
            lax.fori_loop(0, QUARTER // 16, zero_body, 0)
            for k in range(NPAD // VAL_CHUNK):
                base = k * VAL_CHUNK
                pltpu.sync_copy(
                    vft_hbm.at[ch, pl.ds(base, VAL_CHUNK)], val_v
                )

                def scan_body(w, _, base=base, q=q):
                    off = base + w * 16
                    gidx = idx_v[pl.ds(off, 16)]
                    lidx = gidx - q * QUARTER
                    msk = (lidx >= 0) & (lidx < QUARTER)
                    lidx = jnp.where(msk, lidx, 0)
                    vals = val_v[pl.ds(w * 16, 16)]
                    plsc.store_scatter(qbuf, [lidx], vals, mask=msk)
                    return _

                lax.fori_loop(0, VAL_CHUNK // 16, scan_body, 0)
            pltpu.sync_copy(qbuf, out_hbm.at[ch, pl.ds(q * QUARTER, QUARTER)])


def _sc_scatter(vft, idx):
    mesh = plsc.VectorSubcoreMesh(core_axis_name="c", subcore_axis_name="s")
    return pl.kernel(
        _sc_scatter_body,
        mesh=mesh,
        compiler_params=pltpu.CompilerParams(needs_layout_passes=False),
        out_type=jax.ShapeDtypeStruct((NCH, PLANE), jnp.float32),
        scratch_types=[
            pltpu.VMEM((NPAD,), jnp.int32),
            pltpu.VMEM((VAL_CHUNK,), jnp.float32),
            pltpu.VMEM((QUARTER,), jnp.float32),
        ],
    )(vft, idx)


def kernel(voxel_features, coords):
    coords_t = coords.astype(jnp.int32).T
    coords_t = jnp.pad(coords_t, ((0, 0), (0, NPAD - NVOX)), constant_values=NXK)
    vf = jnp.pad(voxel_features, ((0, NPAD - NVOX), (0, 0)))
    vft, idx = _tc_prep(vf, coords_t)
    canvas = _sc_scatter(vft, idx)
    return canvas.reshape(1, NCH, NXK, NYK)


# speedup vs baseline: 1.7292x; 1.5532x over previous
import jax
import jax.numpy as jnp
from jax import lax
from jax.experimental import pallas as pl
from jax.experimental.pallas import tpu as pltpu
from jax.experimental.pallas import tpu_sc as plsc

NXK = 512
NYK = 512
NCH = 64
NVOX = 40000
NPAD = 40960
PLANE = NXK * NYK
NVPT = NPAD // 32
POSPT = PLANE // 32
TBLK = 8192
CT_ROWS = PLANE + TBLK


def _tc_idx_body(ct_ref, idx_ref):
    i = pl.program_id(0)
    ct = ct_ref[:, pl.ds(i * 2048, 2048)]
    idx_ref[...] = ct[0, :] * NYK + ct[1, :]


def _tc_idx(coords_t):
    return pl.pallas_call(
        _tc_idx_body,
        grid=(20,),
        in_specs=[pl.BlockSpec((2, NPAD), lambda i: (0, 0))],
        out_specs=pl.BlockSpec((2048,), lambda i: (i,)),
        out_shape=jax.ShapeDtypeStruct((NPAD,), jnp.int32),
    )(coords_t)


def _sc_idplane_body(idx_hbm, idp_hbm, idx_v, idb):
    wid = lax.axis_index("s") * 2 + lax.axis_index("c")
    base = wid * POSPT

    pltpu.sync_copy(idx_hbm, idx_v)

    def zero_body(i, c):
        idb[pl.ds(i * 16, 16)] = jnp.zeros((16,), jnp.int32)
        return c

    lax.fori_loop(0, POSPT // 16, zero_body, 0)

    def scan_body(w, c):
        gidx = idx_v[pl.ds(w * 16, 16)]
        lpos = gidx - base
        msk = (lpos >= 0) & (lpos < POSPT)
        lpos = jnp.where(msk, lpos, 0)
        ids = lax.iota(jnp.int32, 16) + (w * 16 + 1)
        plsc.store_scatter(idb, [lpos], ids, mask=msk)
        return c

    lax.fori_loop(0, NPAD // 16, scan_body, 0)
    pltpu.sync_copy(idb, idp_hbm.at[pl.ds(base, POSPT)])


def _sc_idplane(idx):
    mesh = plsc.VectorSubcoreMesh(core_axis_name="c", subcore_axis_name="s")
    return pl.kernel(
        _sc_idplane_body,
        mesh=mesh,
        compiler_params=pltpu.CompilerParams(needs_layout_passes=False),
        out_type=jax.ShapeDtypeStruct((PLANE,), jnp.int32),
        scratch_types=[
            pltpu.VMEM((NPAD,), jnp.int32),
            pltpu.VMEM((POSPT,), jnp.int32),
        ],
    )(idx)


def _sc_rows_body(vf_hbm, idx_hbm, idp_hbm, ct_hbm, rows, idxb, winb, tgtb,
                  sem):
    sc = lax.axis_index("c")
    t = lax.axis_index("s")
    wid = t * 2 + sc
    vbase = wid * NVPT

    pltpu.sync_copy(idx_hbm.at[pl.ds(vbase, NVPT)], idxb)

    def clamp_body(w, c):
        g = idxb[pl.ds(w * 16, 16)]
        idxb[pl.ds(w * 16, 16)] = jnp.minimum(g, PLANE - 1)
        return c

    lax.fori_loop(0, NVPT // 16, clamp_body, 0)
    pltpu.async_copy(idp_hbm.at[idxb], winb, sem).wait()

    def tgt_body(w, c):
        g = idxb[pl.ds(w * 16, 16)]
        wi = winb[pl.ds(w * 16, 16)]
        vg = lax.iota(jnp.int32, 16) + (vbase + w * 16 + 1)
        tgtb[pl.ds(w * 16, 16)] = jnp.where(
            wi == vg, g, PLANE + lax.iota(jnp.int32, 16)
        )
        return c

    lax.fori_loop(0, NVPT // 16, tgt_body, 0)

    tail = NVOX - (NPAD - NVPT)

    @pl.when(vbase + NVPT <= NVOX)
    def _():
        pltpu.sync_copy(vf_hbm.at[pl.ds(vbase, NVPT)], rows)

    @pl.when(vbase + NVPT > NVOX)
    def _():
        pltpu.sync_copy(
            vf_hbm.at[pl.ds(vbase, tail)], rows.at[pl.ds(0, tail)]
        )

    pltpu.async_copy(rows, ct_hbm.at[tgtb], sem).wait()


def _sc_rows(vf, idx, idp):
    mesh = plsc.VectorSubcoreMesh(core_axis_name="c", subcore_axis_name="s")
    return pl.kernel(
        _sc_rows_body,
        mesh=mesh,
        compiler_params=pltpu.CompilerParams(
            needs_layout_passes=False, use_tc_tiling_on_sc=False),
        out_type=jax.ShapeDtypeStruct((CT_ROWS, NCH), jnp.float32),
        scratch_types=[
            pltpu.VMEM((NVPT, NCH), jnp.float32),
            pltpu.VMEM((NVPT,), jnp.int32),
            pltpu.VMEM((NVPT,), jnp.int32),
            pltpu.VMEM((NVPT,), jnp.int32),
            pltpu.SemaphoreType.DMA,
        ],
    )(vf, idx, idp)


def _tc_paint_body(ct_ref, idp_ref, out_ref):
    m = (idp_ref[...] > 0)[None, :]
    out_ref[...] = jnp.where(m, ct_ref[...].T, 0.0)


def _tc_paint(ct, idp):
    return pl.pallas_call(
        _tc_paint_body,
        grid=(PLANE // TBLK,),
        in_specs=[
            pl.BlockSpec((TBLK, NCH), lambda i: (i, 0)),
            pl.BlockSpec((TBLK,), lambda i: (i,)),
        ],
        out_specs=pl.BlockSpec((NCH, TBLK), lambda i: (0, i)),
        out_shape=jax.ShapeDtypeStruct((NCH, PLANE), jnp.float32),
    )(ct, idp)


def kernel(voxel_features, coords):
    coords_t = coords.astype(jnp.int32).T
    coords_t = jnp.pad(coords_t, ((0, 0), (0, NPAD - NVOX)),
                       constant_values=NXK)
    idx = _tc_idx(coords_t)
    idp = _sc_idplane(idx)
    ct = _sc_rows(voxel_features, idx, idp)
    canvas = _tc_paint(ct, idp)
    return canvas.reshape(1, NCH, NXK, NYK)
